# baseline (speedup 1.0000x reference)
import jax
import jax.numpy as jnp
from jax import lax
from jax.experimental import pallas as pl
from jax.experimental.pallas import tpu as pltpu
from jax.experimental.pallas import tpu_sc as plsc

_B, _S, _D = 4, 2048, 768
_E, _TOPK, _F = 16, 2, 3072
_N = _B * _S
_TB = 256
_TPT = 256
_SEGTOK = _N // 2
_SEGCAP = _SEGTOK + _TB
_SEGBLK = _SEGCAP // _TB
_GROWS = 2 * _SEGCAP
_GW = _D + 128


def _gate_kernel(x_ref, wg_ref, bg_ref, c_ref):
    logits = jax.lax.dot_general(
        wg_ref[...], x_ref[...], (((0,), (1,)), ((), ())),
        preferred_element_type=jnp.float32) + bg_ref[...]
    m = jnp.max(logits, axis=1, keepdims=True)
    ex = jnp.exp(logits - m)
    w = ex / jnp.sum(ex, axis=1, keepdims=True)
    w0 = w[0:1, :]
    w1 = w[1:2, :]
    gt0 = jnp.sum((w > w0).astype(jnp.int32), axis=0, keepdims=True)
    gt1 = (jnp.sum((w > w1).astype(jnp.int32), axis=0, keepdims=True)
           + (w0 == w1).astype(jnp.int32))
    c0 = jnp.where(gt0 < _TOPK, w0, 0.0)
    c1 = jnp.where(gt1 < _TOPK, w1, 0.0)
    row = jax.lax.broadcasted_iota(jnp.int32, (8, x_ref.shape[0]), 0)
    c_ref[...] = jnp.where(row == 0, c0, jnp.where(row == 1, c1, 0.0))


def _dispatch_kernel(x_hbm, c0_hbm, c1_hbm, gx, segcnt, tilecnt,
                     c0v, c1v, cv, cntv, xbuf, counts_sh, sem, fsem):
    cidx = lax.axis_index("c")
    sidx = lax.axis_index("s")
    wid = sidx * 2 + cidx
    tb = wid * _TPT
    seg_base = cidx * _SEGCAP
    lane = lax.iota(jnp.int32, 16)
    zero16 = jnp.zeros((16,), jnp.int32)

    pltpu.sync_copy(c0_hbm.at[pl.ds(tb, _TPT)], c0v)
    pltpu.sync_copy(c1_hbm.at[pl.ds(tb, _TPT)], c1v)

    def count_body(q, cnt):
        v0 = c0v[pl.ds(q * 16, 16)]
        v1 = c1v[pl.ds(q * 16, 16)]
        for l in range(16):
            cond = (v0[l] > 0.0) | (v1[l] > 0.0)
            cnt = cnt + jnp.where(cond, 1, 0)
        return cnt

    cnt = lax.fori_loop(0, _TPT // 16, count_body, jnp.int32(0))

    cntv[...] = zero16 + cnt
    pltpu.sync_copy(cntv, tilecnt.at[wid])
    pltpu.sync_copy(cntv, counts_sh.at[pl.ds(sidx * 16, 16)])
    plsc.subcore_barrier()
    pltpu.sync_copy(counts_sh, cv)

    def pfx(r, carry):
        base, tot = carry
        padded = ((cv[pl.ds(r * 16, 16)][0] + 15) // 16) * 16
        return base + jnp.where(r < sidx, padded, 0), tot + padded

    base_local, m_c = lax.fori_loop(0, 16, pfx, (jnp.int32(0), jnp.int32(0)))
    base = seg_base + base_local

    @pl.when(sidx == 0)
    def _():
        cntv[...] = zero16 + m_c
        pltpu.sync_copy(cntv, segcnt.at[cidx])

    zf32 = jnp.zeros((16,), jnp.float32)

    def chunk_body(q, run):
        v0 = c0v[pl.ds(q * 16, 16)]
        v1 = c1v[pl.ds(q * 16, 16)]
        for l in range(16):
            pltpu.async_copy(x_hbm.at[tb + q * 16 + l],
                             xbuf.at[pl.ds(l * _GW, _D)], sem)
        for l in range(16):
            pltpu.make_async_copy(x_hbm.at[0],
                                  xbuf.at[pl.ds(l * _GW, _D)], sem).wait()
        for l in range(16):
            cpiece = jnp.where(lane == 0, zf32 + v0[l],
                               jnp.where(lane == 1, zf32 + v1[l], zf32))
            xbuf[pl.ds(l * _GW + _D, 16)] = cpiece
        nrun = run
        for l in range(16):
            cond = (v0[l] > 0.0) | (v1[l] > 0.0)

            @pl.when(cond)
            def _(l=l, nrun=nrun):
                pltpu.async_copy(xbuf.at[pl.ds(l * _GW, _GW)],
                                 gx.at[base + nrun], fsem)
            nrun = nrun + jnp.where(cond, 1, 0)
        for l in range(16):
            cond = (v0[l] > 0.0) | (v1[l] > 0.0)

            @pl.when(cond)
            def _(l=l):
                pltpu.make_async_copy(xbuf.at[pl.ds(l * _GW, _GW)],
                                      gx.at[0], fsem).wait()
        return nrun

    lax.fori_loop(0, _TPT // 16, chunk_body, jnp.int32(0))


def _ffn_kernel(cnt_ref, gx_ref, w1_ref, b1_ref, w2_ref, b2_ref, gy_ref):
    i = pl.program_id(0)
    seg = i // 16
    j = i % 16

    @pl.when(j * _TB < cnt_ref[seg])
    def _():
        blk = gx_ref[...]
        x = blk[:, :_D].astype(jnp.bfloat16)
        c0 = blk[:, _D:_D + 1]
        c1 = blk[:, _D + 1:_D + 2]
        acc = jnp.zeros((_TB, _D), jnp.float32)
        for e in range(2):
            h = jax.lax.dot_general(
                x, w1_ref[e], (((1,), (0,)), ((), ())),
                preferred_element_type=jnp.float32) + b1_ref[e]
            h = h * 0.5 * (1.0 + jax.lax.erf(h * 0.7071067811865476))
            o = jax.lax.dot_general(
                h.astype(jnp.bfloat16), w2_ref[e], (((1,), (0,)), ((), ())),
                preferred_element_type=jnp.float32) + b2_ref[e]
            m = jnp.max(o, axis=1, keepdims=True)
            p = jnp.exp(o - m)
            o = p / jnp.sum(p, axis=1, keepdims=True)
            acc = acc + (c0 if e == 0 else c1) * o
        gy_ref[...] = acc


def _combine_kernel(gy, c0_hbm, c1_hbm, tilecnt, zrow, out,
                    c0v, c1v, cv, sem):
    cidx = lax.axis_index("c")
    sidx = lax.axis_index("s")
    wid = sidx * 2 + cidx
    tb = wid * _TPT
    seg_base = cidx * _SEGCAP

    pltpu.sync_copy(c0_hbm.at[pl.ds(tb, _TPT)], c0v)
    pltpu.sync_copy(c1_hbm.at[pl.ds(tb, _TPT)], c1v)

    def pfx2(r, base):
        pltpu.sync_copy(tilecnt.at[r * 2 + cidx], cv)
        padded = ((cv[...][0] + 15) // 16) * 16
        return base + jnp.where(r < sidx, padded, 0)

    base_local = lax.fori_loop(0, 16, pfx2, jnp.int32(0))
    base = seg_base + base_local

    def chunk_body(q, run):
        v0 = c0v[pl.ds(q * 16, 16)]
        v1 = c1v[pl.ds(q * 16, 16)]
        nrun = run
        for l in range(16):
            cond = (v0[l] > 0.0) | (v1[l] > 0.0)
            t = tb + q * 16 + l

            @pl.when(cond)
            def _(t=t, nrun=nrun):
                pltpu.async_copy(gy.at[base + nrun], out.at[t], sem)

            @pl.when(jnp.logical_not(cond))
            def _(t=t):
                pltpu.async_copy(zrow.at[wid], out.at[t], sem)
            nrun = nrun + jnp.where(cond, 1, 0)
        return nrun

    lax.fori_loop(0, _TPT // 16, chunk_body, jnp.int32(0))

    def drain(q, _):
        pltpu.make_async_copy(zrow.at[0], out.at[0], sem).wait()
        return 0

    lax.fori_loop(0, _TPT, drain, 0)


def kernel(x, Wg, bg, W1, b1, W2, b2):
    Bx, Sx, Dx = x.shape
    n = Bx * Sx
    x_f = x.reshape(n, Dx)

    c2 = pl.pallas_call(
        _gate_kernel,
        grid=(Bx,),
        in_specs=[
            pl.BlockSpec((Sx, Dx), lambda i: (i, 0)),
            pl.BlockSpec((Dx, _E), lambda i: (0, 0)),
            pl.BlockSpec((_E, 1), lambda i: (0, 0)),
        ],
        out_specs=pl.BlockSpec((8, Sx), lambda i: (0, i)),
        out_shape=jax.ShapeDtypeStruct((8, n), jnp.float32),
    )(x_f, Wg, bg.reshape(_E, 1))
    c0 = c2[0]
    c1 = c2[1]

    mesh = plsc.VectorSubcoreMesh(core_axis_name="c", subcore_axis_name="s",
                                  num_cores=2, num_subcores=16)
    gx, segcnt, tilecnt = pl.kernel(
        _dispatch_kernel,
        out_type=(
            jax.ShapeDtypeStruct((_GROWS, _GW), jnp.float32),
            jax.ShapeDtypeStruct((2, 16), jnp.int32),
            jax.ShapeDtypeStruct((32, 16), jnp.int32),
        ),
        mesh=mesh,
        scratch_types=[
            pltpu.VMEM((_TPT,), jnp.float32),
            pltpu.VMEM((_TPT,), jnp.float32),
            pltpu.VMEM((256,), jnp.int32),
            pltpu.VMEM((16,), jnp.int32),
            pltpu.VMEM((16 * _GW,), jnp.float32),
            pltpu.VMEM_SHARED((256,), jnp.int32),
            pltpu.SemaphoreType.DMA,
            pltpu.SemaphoreType.DMA,
        ],
    )(x_f, c0, c1)

    counts = segcnt[:, 0]

    def _rowmap(i, cnt):
        return ((i // 16) * _SEGBLK + jnp.minimum(
            i % 16, jnp.maximum(cnt[i // 16] - 1, 0) // _TB), 0)

    gy = pl.pallas_call(
        _ffn_kernel,
        grid_spec=pltpu.PrefetchScalarGridSpec(
            num_scalar_prefetch=1,
            grid=(32,),
            in_specs=[
                pl.BlockSpec((_TB, _GW), _rowmap),
                pl.BlockSpec((2, Dx, _F), lambda i, cnt: (0, 0, 0)),
                pl.BlockSpec((2, _F), lambda i, cnt: (0, 0)),
                pl.BlockSpec((2, _F, Dx), lambda i, cnt: (0, 0, 0)),
                pl.BlockSpec((2, Dx), lambda i, cnt: (0, 0)),
            ],
            out_specs=pl.BlockSpec((_TB, Dx), _rowmap),
        ),
        out_shape=jax.ShapeDtypeStruct((_GROWS, Dx), jnp.float32),
    )(counts, gx, W1[:2].astype(jnp.bfloat16), b1[:2],
      W2[:2].astype(jnp.bfloat16), b2[:2])

    out_full = pl.kernel(
        _combine_kernel,
        out_type=jax.ShapeDtypeStruct((n, Dx), jnp.float32),
        mesh=mesh,
        scratch_types=[
            pltpu.VMEM((_TPT,), jnp.float32),
            pltpu.VMEM((_TPT,), jnp.float32),
            pltpu.VMEM((16,), jnp.int32),
            pltpu.SemaphoreType.DMA,
        ],
    )(gy, c0, c1, tilecnt, jnp.zeros((32, Dx), jnp.float32))

    return out_full.reshape(Bx, Sx, Dx)


# speedup vs baseline: 3.8435x; 3.8435x over previous
import jax
import jax.numpy as jnp
from jax.experimental import pallas as pl
from jax.experimental.pallas import tpu as pltpu

_B, _S, _D = 4, 2048, 768
_E, _TOPK, _F = 16, 2, 3072
_TB = 256
_CW = 128


def _gate_kernel(x_ref, wg_ref, bg_ref, c_ref):
    logits = jax.lax.dot_general(
        x_ref[...], wg_ref[...], (((1,), (0,)), ((), ())),
        preferred_element_type=jnp.float32) + bg_ref[...]
    m = jnp.max(logits, axis=0, keepdims=True)
    ex = jnp.exp(logits - m)
    w = ex / jnp.sum(ex, axis=0, keepdims=True)
    w0 = w[:, 0:1]
    w1 = w[:, 1:2]
    gt0 = jnp.sum((w > w0).astype(jnp.int32), axis=1, keepdims=True)
    gt1 = (jnp.sum((w > w1).astype(jnp.int32), axis=1, keepdims=True)
           + (w0 == w1).astype(jnp.int32))
    c0 = jnp.where(gt0 < _TOPK, w0, 0.0)
    c1 = jnp.where(gt1 < _TOPK, w1, 0.0)
    col = jax.lax.broadcasted_iota(jnp.int32, (x_ref.shape[0], _CW), 1)
    c_ref[...] = jnp.where(col == 0, c0, jnp.where(col == 1, c1, 0.0))


def _ffn_kernel(x_ref, c_ref, w1_ref, b1_ref, w2_ref, b2_ref, o_ref):
    x = x_ref[...].astype(jnp.bfloat16)
    acc = jnp.zeros((x.shape[0], _D), jnp.float32)
    for e in range(2):
        h = jax.lax.dot_general(
            x, w1_ref[e], (((1,), (0,)), ((), ())),
            preferred_element_type=jnp.float32) + b1_ref[e]
        h = h * 0.5 * (1.0 + jax.lax.erf(h * 0.7071067811865476))
        o = jax.lax.dot_general(
            h.astype(jnp.bfloat16), w2_ref[e], (((1,), (0,)), ((), ())),
            preferred_element_type=jnp.float32) + b2_ref[e]
        m = jnp.max(o, axis=1, keepdims=True)
        p = jnp.exp(o - m)
        o = p / jnp.sum(p, axis=1, keepdims=True)
        acc = acc + c_ref[:, e:e + 1] * o
    o_ref[...] = acc


def kernel(x, Wg, bg, W1, b1, W2, b2):
    Bx, Sx, Dx = x.shape
    n = Bx * Sx
    x_f = x.reshape(n, Dx)

    c = pl.pallas_call(
        _gate_kernel,
        grid=(Bx,),
        in_specs=[
            pl.BlockSpec((Sx, Dx), lambda i: (i, 0)),
            pl.BlockSpec((Dx, _E), lambda i: (0, 0)),
            pl.BlockSpec((1, _E), lambda i: (0, 0)),
        ],
        out_specs=pl.BlockSpec((Sx, _CW), lambda i: (i, 0)),
        out_shape=jax.ShapeDtypeStruct((n, _CW), jnp.float32),
    )(x_f, Wg, bg.reshape(1, _E))

    out = pl.pallas_call(
        _ffn_kernel,
        grid=(n // _TB,),
        in_specs=[
            pl.BlockSpec((_TB, Dx), lambda i: (i, 0)),
            pl.BlockSpec((_TB, _CW), lambda i: (i, 0)),
            pl.BlockSpec((2, Dx, _F), lambda i: (0, 0, 0)),
            pl.BlockSpec((2, _F), lambda i: (0, 0)),
            pl.BlockSpec((2, _F, Dx), lambda i: (0, 0, 0)),
            pl.BlockSpec((2, Dx), lambda i: (0, 0)),
        ],
        out_specs=pl.BlockSpec((_TB, Dx), lambda i: (i, 0)),
        out_shape=jax.ShapeDtypeStruct((n, Dx), jnp.float32),
    )(x_f, c, W1[:2].astype(jnp.bfloat16), b1[:2], W2[:2].astype(jnp.bfloat16),
      b2[:2])

    return out.reshape(Bx, Sx, Dx)
